# baseline (speedup 1.0000x reference)
import functools

import jax
import jax.numpy as jnp
from jax import lax
from jax.experimental import pallas as pl
from jax.experimental.pallas import tpu as pltpu
from jax.experimental.pallas import tpu_sc as plsc

NN = 10000
FD = 128
EE = 320000

NROW = 80
NPAD = NROW * 128
NW = 32
CH = 128
NCHUNK = 80
EPW = CH * NCHUNK
EPAD = NW * EPW
EB = 4096
HG = EPAD // EB
RB = 640
GR = NPAD // RB
RPT = NPAD // 16


def _hist_body(s_ref, d_ref, ns_ref, nd_ref, accs, accd):
    i = pl.program_id(0)
    iota_hi = lax.broadcasted_iota(jnp.int32, (1, NROW), 1)
    iota_lo = lax.broadcasted_iota(jnp.int32, (1, 128), 1)

    def onehot_dot(idx):
        hi = lax.shift_right_logical(idx, 7)
        lo = lax.bitwise_and(idx, 127)
        oh = jnp.where(hi == iota_hi, 1.0, 0.0)
        ol = jnp.where(lo == iota_lo, 1.0, 0.0)
        return lax.dot_general(oh, ol, (((0,), (0,)), ((), ())),
                               preferred_element_type=jnp.float32)

    cs = onehot_dot(s_ref[...])
    cd = onehot_dot(d_ref[...])

    @pl.when(i == 0)
    def _():
        accs[...] = cs
        accd[...] = cd

    @pl.when(i > 0)
    def _():
        accs[...] = accs[...] + cs
        accd[...] = accd[...] + cd

    @pl.when(i == HG - 1)
    def _():
        ns_ref[...] = lax.rsqrt(jnp.maximum(accs[...], 1.0))
        nd_ref[...] = lax.rsqrt(jnp.maximum(accd[...], 1.0))


_hist = pl.pallas_call(
    _hist_body,
    grid=(HG,),
    in_specs=[pl.BlockSpec((EB, 1), lambda i: (i, 0)),
              pl.BlockSpec((EB, 1), lambda i: (i, 0))],
    out_specs=[pl.BlockSpec((NROW, 128), lambda i: (0, 0)),
               pl.BlockSpec((NROW, 128), lambda i: (0, 0))],
    out_shape=[jax.ShapeDtypeStruct((NROW, 128), jnp.float32),
               jax.ShapeDtypeStruct((NROW, 128), jnp.float32)],
    scratch_shapes=[pltpu.VMEM((NROW, 128), jnp.float32),
                    pltpu.VMEM((NROW, 128), jnp.float32)],
)


def _mm_scale_body(x_ref, w_ref, ns_ref, o_ref):
    o_ref[...] = jnp.dot(x_ref[...], w_ref[...],
                         preferred_element_type=jnp.float32) * ns_ref[...]


_mm_scale = pl.pallas_call(
    _mm_scale_body,
    grid=(GR,),
    in_specs=[pl.BlockSpec((RB, FD), lambda i: (i, 0)),
              pl.BlockSpec((FD, FD), lambda i: (0, 0)),
              pl.BlockSpec((RB, 1), lambda i: (i, 0))],
    out_specs=pl.BlockSpec((RB, FD), lambda i: (i, 0)),
    out_shape=jax.ShapeDtypeStruct((NPAD, FD), jnp.float32),
)


def _combine_mm_body(a0_ref, a1_ref, nd_ref, b_ref, w_ref, ns_ref, o_ref):
    h = (a0_ref[...] + a1_ref[...]) * nd_ref[...] + b_ref[...]
    h = jnp.maximum(h, 0.0)
    o_ref[...] = jnp.dot(h, w_ref[...],
                         preferred_element_type=jnp.float32) * ns_ref[...]


_combine_mm = pl.pallas_call(
    _combine_mm_body,
    grid=(GR,),
    in_specs=[pl.BlockSpec((RB, FD), lambda i: (i, 0)),
              pl.BlockSpec((RB, FD), lambda i: (i, 0)),
              pl.BlockSpec((RB, 1), lambda i: (i, 0)),
              pl.BlockSpec((1, FD), lambda i: (0, 0)),
              pl.BlockSpec((FD, FD), lambda i: (0, 0)),
              pl.BlockSpec((RB, 1), lambda i: (i, 0))],
    out_specs=pl.BlockSpec((RB, FD), lambda i: (i, 0)),
    out_shape=jax.ShapeDtypeStruct((NPAD, FD), jnp.float32),
)


def _head_body(a0_ref, a1_ref, nd_ref, b_ref, pw_ref, f1w_ref, f1b_ref,
               f2w_ref, f2b_ref, o_ref, mx, sm):
    i = pl.program_id(0)
    h = (a0_ref[...] + a1_ref[...]) * nd_ref[...] + b_ref[...]
    h = jnp.maximum(h, 0.0)
    valid = lax.broadcasted_iota(jnp.int32, (RB, FD), 0) < (NN - i * RB)
    bm = jnp.max(jnp.where(valid, h, -1e30), axis=0, keepdims=True)
    bs = jnp.sum(jnp.where(valid, h, 0.0), axis=0, keepdims=True)

    @pl.when(i == 0)
    def _():
        mx[...] = bm
        sm[...] = bs

    @pl.when(i > 0)
    def _():
        mx[...] = jnp.maximum(mx[...], bm)
        sm[...] = sm[...] + bs

    @pl.when(i == GR - 1)
    def _():
        pwv = pw_ref[...]
        e = jnp.exp(pwv - jnp.max(pwv))
        wgt = e / jnp.sum(e)
        stacked = jnp.concatenate([mx[...], sm[...] * (1.0 / NN)], axis=0)
        rep = jnp.sum(stacked * wgt, axis=0, keepdims=True)
        x1 = jnp.dot(rep, f1w_ref[...],
                     preferred_element_type=jnp.float32) + f1b_ref[...]
        x1 = jnp.maximum(x1, 0.0)
        o_ref[...] = jnp.dot(x1, f2w_ref[...],
                             preferred_element_type=jnp.float32) + f2b_ref[...]


_head = pl.pallas_call(
    _head_body,
    grid=(GR,),
    in_specs=[pl.BlockSpec((RB, FD), lambda i: (i, 0)),
              pl.BlockSpec((RB, FD), lambda i: (i, 0)),
              pl.BlockSpec((RB, 1), lambda i: (i, 0)),
              pl.BlockSpec((1, FD), lambda i: (0, 0)),
              pl.BlockSpec((2, 1), lambda i: (0, 0)),
              pl.BlockSpec((FD, FD), lambda i: (0, 0)),
              pl.BlockSpec((1, FD), lambda i: (0, 0)),
              pl.BlockSpec((FD, 2), lambda i: (0, 0)),
              pl.BlockSpec((1, 2), lambda i: (0, 0))],
    out_specs=pl.BlockSpec((1, 2), lambda i: (0, 0)),
    out_shape=jax.ShapeDtypeStruct((1, 2), jnp.float32),
    scratch_shapes=[pltpu.VMEM((1, FD), jnp.float32),
                    pltpu.VMEM((1, FD), jnp.float32)],
)


_mesh = plsc.VectorSubcoreMesh(core_axis_name="c", subcore_axis_name="s")


@functools.partial(
    pl.kernel,
    out_type=jax.ShapeDtypeStruct((2, NPAD, FD), jnp.float32),
    mesh=_mesh,
    scratch_types=[
        pltpu.VMEM((CH,), jnp.int32),
        pltpu.VMEM((CH,), jnp.int32),
        pltpu.VMEM((CH, FD), jnp.float32),
        pltpu.VMEM((NROW, FD), jnp.float32),
        pltpu.VMEM_SHARED((NPAD, FD), jnp.float32),
        pltpu.SemaphoreType.DMA,
    ],
)
def _spmm(y_hbm, src_hbm, dst_hbm, z_hbm, out_hbm,
          sidx, didx, rows, zbuf, agg_sh, sem):
    cid = lax.axis_index("c")
    sid = lax.axis_index("s")
    wid = sid * 2 + cid

    pltpu.sync_copy(z_hbm, zbuf)
    r0 = sid * RPT
    for k in range(RPT // NROW):
        pltpu.sync_copy(zbuf, agg_sh.at[pl.ds(r0 + k * NROW, NROW)])
    plsc.subcore_barrier()

    ebase = wid * EPW

    def step(t, carry):
        b = ebase + t * CH
        pltpu.sync_copy(src_hbm.at[pl.ds(b, CH)], sidx)
        pltpu.sync_copy(dst_hbm.at[pl.ds(b, CH)], didx)
        pltpu.async_copy(y_hbm.at[sidx], rows, sem).wait()
        pltpu.sync_copy(rows, agg_sh.at[didx], add=True)
        return carry

    lax.fori_loop(0, NCHUNK, step, 0)
    plsc.subcore_barrier()

    pltpu.sync_copy(agg_sh.at[pl.ds(r0, RPT)],
                    out_hbm.at[cid, pl.ds(r0, RPT)])


def kernel(features, edge_index, W1, b1, W2, b2, W3, b3, pool_weight,
           fc1_W, fc1_b, fc2_W, fc2_b):
    src = edge_index[0]
    dst = edge_index[1]
    padv = jnp.full((EPAD - EE,), NN, jnp.int32)
    srcp = jnp.concatenate([src, padv])
    dstp = jnp.concatenate([dst, padv])

    fpad = jnp.pad(features, ((0, NPAD - NN), (0, 0)))
    zeros_tile = jnp.zeros((NROW, FD), jnp.float32)

    ns2d, nd2d = _hist(srcp.reshape(EPAD, 1), dstp.reshape(EPAD, 1))
    ns = ns2d.reshape(NPAD, 1)
    nd = nd2d.reshape(NPAD, 1)

    y = _mm_scale(fpad, W1, ns)
    agg = _spmm(y, srcp, dstp, zeros_tile)
    y = _combine_mm(agg[0], agg[1], nd, b1.reshape(1, FD), W2, ns)
    agg = _spmm(y, srcp, dstp, zeros_tile)
    y = _combine_mm(agg[0], agg[1], nd, b2.reshape(1, FD), W3, ns)
    agg = _spmm(y, srcp, dstp, zeros_tile)
    out = _head(agg[0], agg[1], nd, b3.reshape(1, FD),
                pool_weight.reshape(2, 1), fc1_W, fc1_b.reshape(1, FD),
                fc2_W, fc2_b.reshape(1, 2))
    return out


# speedup vs baseline: 1.8983x; 1.8983x over previous
import functools

import jax
import jax.numpy as jnp
from jax import lax
from jax.experimental import pallas as pl
from jax.experimental.pallas import tpu as pltpu
from jax.experimental.pallas import tpu_sc as plsc

NN = 10000
FD = 128
EE = 320000

NROW = 80
NPAD = NROW * 128
NW = 32
CH = 128
NCHUNK = 79
EPW = CH * NCHUNK
EPAD = NW * EPW
EB = 4096
HG = EPAD // EB
RB = 640
GR = NPAD // RB
RPT = NPAD // 16


def _hist_body(s_ref, d_ref, ns_ref, nd_ref, accs, accd):
    i = pl.program_id(0)
    iota_hi = lax.broadcasted_iota(jnp.int32, (1, NROW), 1)
    iota_lo = lax.broadcasted_iota(jnp.int32, (1, 128), 1)

    def onehot_dot(idx):
        hi = lax.shift_right_logical(idx, 7)
        lo = lax.bitwise_and(idx, 127)
        oh = jnp.where(hi == iota_hi, 1.0, 0.0)
        ol = jnp.where(lo == iota_lo, 1.0, 0.0)
        return lax.dot_general(oh, ol, (((0,), (0,)), ((), ())),
                               preferred_element_type=jnp.float32)

    cs = onehot_dot(s_ref[...])
    cd = onehot_dot(d_ref[...])

    @pl.when(i == 0)
    def _():
        accs[...] = cs
        accd[...] = cd

    @pl.when(i > 0)
    def _():
        accs[...] = accs[...] + cs
        accd[...] = accd[...] + cd

    @pl.when(i == HG - 1)
    def _():
        ns_ref[...] = lax.rsqrt(jnp.maximum(accs[...], 1.0))
        nd_ref[...] = lax.rsqrt(jnp.maximum(accd[...], 1.0))


_hist = pl.pallas_call(
    _hist_body,
    grid=(HG,),
    in_specs=[pl.BlockSpec((EB, 1), lambda i: (i, 0)),
              pl.BlockSpec((EB, 1), lambda i: (i, 0))],
    out_specs=[pl.BlockSpec((NROW, 128), lambda i: (0, 0)),
               pl.BlockSpec((NROW, 128), lambda i: (0, 0))],
    out_shape=[jax.ShapeDtypeStruct((NROW, 128), jnp.float32),
               jax.ShapeDtypeStruct((NROW, 128), jnp.float32)],
    scratch_shapes=[pltpu.VMEM((NROW, 128), jnp.float32),
                    pltpu.VMEM((NROW, 128), jnp.float32)],
)


def _mm_scale_body(x_ref, w_ref, ns_ref, o_ref):
    o_ref[...] = jnp.dot(x_ref[...], w_ref[...],
                         preferred_element_type=jnp.float32) * ns_ref[...]


_mm_scale = pl.pallas_call(
    _mm_scale_body,
    grid=(GR,),
    in_specs=[pl.BlockSpec((RB, FD), lambda i: (i, 0)),
              pl.BlockSpec((FD, FD), lambda i: (0, 0)),
              pl.BlockSpec((RB, 1), lambda i: (i, 0))],
    out_specs=pl.BlockSpec((RB, FD), lambda i: (i, 0)),
    out_shape=jax.ShapeDtypeStruct((NPAD, FD), jnp.float32),
)


def _combine_mm_body(a0_ref, a1_ref, nd_ref, b_ref, w_ref, ns_ref, o_ref):
    h = (a0_ref[...] + a1_ref[...]) * nd_ref[...] + b_ref[...]
    h = jnp.maximum(h, 0.0)
    o_ref[...] = jnp.dot(h, w_ref[...],
                         preferred_element_type=jnp.float32) * ns_ref[...]


_combine_mm = pl.pallas_call(
    _combine_mm_body,
    grid=(GR,),
    in_specs=[pl.BlockSpec((RB, FD), lambda i: (i, 0)),
              pl.BlockSpec((RB, FD), lambda i: (i, 0)),
              pl.BlockSpec((RB, 1), lambda i: (i, 0)),
              pl.BlockSpec((1, FD), lambda i: (0, 0)),
              pl.BlockSpec((FD, FD), lambda i: (0, 0)),
              pl.BlockSpec((RB, 1), lambda i: (i, 0))],
    out_specs=pl.BlockSpec((RB, FD), lambda i: (i, 0)),
    out_shape=jax.ShapeDtypeStruct((NPAD, FD), jnp.float32),
)


def _head_body(a0_ref, a1_ref, nd_ref, b_ref, pw_ref, f1w_ref, f1b_ref,
               f2w_ref, f2b_ref, o_ref, mx, sm):
    i = pl.program_id(0)
    h = (a0_ref[...] + a1_ref[...]) * nd_ref[...] + b_ref[...]
    h = jnp.maximum(h, 0.0)
    valid = lax.broadcasted_iota(jnp.int32, (RB, FD), 0) < (NN - i * RB)
    bm = jnp.max(jnp.where(valid, h, -1e30), axis=0, keepdims=True)
    bs = jnp.sum(jnp.where(valid, h, 0.0), axis=0, keepdims=True)

    @pl.when(i == 0)
    def _():
        mx[...] = bm
        sm[...] = bs

    @pl.when(i > 0)
    def _():
        mx[...] = jnp.maximum(mx[...], bm)
        sm[...] = sm[...] + bs

    @pl.when(i == GR - 1)
    def _():
        pwv = pw_ref[...]
        e = jnp.exp(pwv - jnp.max(pwv))
        wgt = e / jnp.sum(e)
        stacked = jnp.concatenate([mx[...], sm[...] * (1.0 / NN)], axis=0)
        rep = jnp.sum(stacked * wgt, axis=0, keepdims=True)
        x1 = jnp.dot(rep, f1w_ref[...],
                     preferred_element_type=jnp.float32) + f1b_ref[...]
        x1 = jnp.maximum(x1, 0.0)
        o_ref[...] = jnp.dot(x1, f2w_ref[...],
                             preferred_element_type=jnp.float32) + f2b_ref[...]


_head = pl.pallas_call(
    _head_body,
    grid=(GR,),
    in_specs=[pl.BlockSpec((RB, FD), lambda i: (i, 0)),
              pl.BlockSpec((RB, FD), lambda i: (i, 0)),
              pl.BlockSpec((RB, 1), lambda i: (i, 0)),
              pl.BlockSpec((1, FD), lambda i: (0, 0)),
              pl.BlockSpec((2, 1), lambda i: (0, 0)),
              pl.BlockSpec((FD, FD), lambda i: (0, 0)),
              pl.BlockSpec((1, FD), lambda i: (0, 0)),
              pl.BlockSpec((FD, 2), lambda i: (0, 0)),
              pl.BlockSpec((1, 2), lambda i: (0, 0))],
    out_specs=pl.BlockSpec((1, 2), lambda i: (0, 0)),
    out_shape=jax.ShapeDtypeStruct((1, 2), jnp.float32),
    scratch_shapes=[pltpu.VMEM((1, FD), jnp.float32),
                    pltpu.VMEM((1, FD), jnp.float32)],
)


_mesh = plsc.VectorSubcoreMesh(core_axis_name="c", subcore_axis_name="s")


@functools.partial(
    pl.kernel,
    out_type=jax.ShapeDtypeStruct((2, NPAD, FD), jnp.float32),
    mesh=_mesh,
    scratch_types=[
        pltpu.VMEM((CH,), jnp.int32),
        pltpu.VMEM((CH,), jnp.int32),
        pltpu.VMEM((CH, FD), jnp.float32),
        pltpu.VMEM((NROW, FD), jnp.float32),
        pltpu.VMEM_SHARED((NPAD, FD), jnp.float32),
        pltpu.SemaphoreType.DMA,
    ],
)
def _spmm(y_hbm, src_hbm, dst_hbm, z_hbm, out_hbm,
          sidx, didx, rows, zbuf, agg_sh, sem):
    cid = lax.axis_index("c")
    sid = lax.axis_index("s")
    wid = sid * 2 + cid

    pltpu.sync_copy(z_hbm, zbuf)
    r0 = sid * RPT
    for k in range(RPT // NROW):
        pltpu.sync_copy(zbuf, agg_sh.at[pl.ds(r0 + k * NROW, NROW)])
    plsc.subcore_barrier()

    ebase = wid * EPW

    def step(t, carry):
        b = ebase + t * CH
        pltpu.sync_copy(src_hbm.at[pl.ds(b, CH)], sidx)
        pltpu.sync_copy(dst_hbm.at[pl.ds(b, CH)], didx)
        pltpu.async_copy(y_hbm.at[sidx], rows, sem).wait()
        pltpu.sync_copy(rows, agg_sh.at[didx], add=True)
        return carry

    lax.fori_loop(0, NCHUNK, step, 0)
    plsc.subcore_barrier()

    pltpu.sync_copy(agg_sh.at[pl.ds(r0, RPT)],
                    out_hbm.at[cid, pl.ds(r0, RPT)])


def kernel(features, edge_index, W1, b1, W2, b2, W3, b3, pool_weight,
           fc1_W, fc1_b, fc2_W, fc2_b):
    src = edge_index[0]
    dst = edge_index[1]
    padv = NN + (jnp.arange(EPAD - EE, dtype=jnp.int32) % (NPAD - NN))
    srcp = jnp.concatenate([src, padv])
    dstp = jnp.concatenate([dst, padv])

    fpad = jnp.pad(features, ((0, NPAD - NN), (0, 0)))
    zeros_tile = jnp.zeros((NROW, FD), jnp.float32)

    ns2d, nd2d = _hist(srcp.reshape(EPAD, 1), dstp.reshape(EPAD, 1))
    ns = ns2d.reshape(NPAD, 1)
    nd = nd2d.reshape(NPAD, 1)

    y = _mm_scale(fpad, W1, ns)
    agg = _spmm(y, srcp, dstp, zeros_tile)
    y = _combine_mm(agg[0], agg[1], nd, b1.reshape(1, FD), W2, ns)
    agg = _spmm(y, srcp, dstp, zeros_tile)
    y = _combine_mm(agg[0], agg[1], nd, b2.reshape(1, FD), W3, ns)
    agg = _spmm(y, srcp, dstp, zeros_tile)
    out = _head(agg[0], agg[1], nd, b3.reshape(1, FD),
                pool_weight.reshape(2, 1), fc1_W, fc1_b.reshape(1, FD),
                fc2_W, fc2_b.reshape(1, 2))
    return out


# speedup vs baseline: 2.7023x; 1.4236x over previous
import functools

import jax
import jax.numpy as jnp
from jax import lax
from jax.experimental import pallas as pl
from jax.experimental.pallas import tpu as pltpu
from jax.experimental.pallas import tpu_sc as plsc

NN = 10000
FD = 128
EE = 320000

NROW = 80
NPAD = NROW * 128
NW = 32
CH = 128
NCHUNK = 80
EPW = CH * NCHUNK
EPAD = NW * EPW
SEG = 16
EB = 4096
HG = EPAD // EB
RB = 640
GR = NPAD // RB
RPT = NPAD // 16


def _hist_body(s_ref, d_ref, ns_ref, nd_ref, accs, accd):
    i = pl.program_id(0)
    iota_hi = lax.broadcasted_iota(jnp.int32, (1, NROW), 1)
    iota_lo = lax.broadcasted_iota(jnp.int32, (1, 128), 1)

    def onehot_dot(idx):
        hi = lax.shift_right_logical(idx, 7)
        lo = lax.bitwise_and(idx, 127)
        oh = jnp.where(hi == iota_hi, 1.0, 0.0)
        ol = jnp.where(lo == iota_lo, 1.0, 0.0)
        return lax.dot_general(oh, ol, (((0,), (0,)), ((), ())),
                               preferred_element_type=jnp.float32)

    cs = onehot_dot(s_ref[...])
    cd = onehot_dot(d_ref[...])

    @pl.when(i == 0)
    def _():
        accs[...] = cs
        accd[...] = cd

    @pl.when(i > 0)
    def _():
        accs[...] = accs[...] + cs
        accd[...] = accd[...] + cd

    @pl.when(i == HG - 1)
    def _():
        ns_ref[...] = lax.rsqrt(jnp.maximum(accs[...], 1.0))
        nd_ref[...] = lax.rsqrt(jnp.maximum(accd[...], 1.0))


_hist = pl.pallas_call(
    _hist_body,
    grid=(HG,),
    in_specs=[pl.BlockSpec((EB, 1), lambda i: (i, 0)),
              pl.BlockSpec((EB, 1), lambda i: (i, 0))],
    out_specs=[pl.BlockSpec((NROW, 128), lambda i: (0, 0)),
               pl.BlockSpec((NROW, 128), lambda i: (0, 0))],
    out_shape=[jax.ShapeDtypeStruct((NROW, 128), jnp.float32),
               jax.ShapeDtypeStruct((NROW, 128), jnp.float32)],
    scratch_shapes=[pltpu.VMEM((NROW, 128), jnp.float32),
                    pltpu.VMEM((NROW, 128), jnp.float32)],
)


def _mm_scale_body(x_ref, w_ref, ns_ref, o_ref):
    o_ref[...] = jnp.dot(x_ref[...], w_ref[...],
                         preferred_element_type=jnp.float32) * ns_ref[...]


_mm_scale = pl.pallas_call(
    _mm_scale_body,
    grid=(GR,),
    in_specs=[pl.BlockSpec((RB, FD), lambda i: (i, 0)),
              pl.BlockSpec((FD, FD), lambda i: (0, 0)),
              pl.BlockSpec((RB, 1), lambda i: (i, 0))],
    out_specs=pl.BlockSpec((RB, FD), lambda i: (i, 0)),
    out_shape=jax.ShapeDtypeStruct((NPAD, FD), jnp.float32),
)


def _combine_mm_body(a0_ref, a1_ref, nd_ref, b_ref, w_ref, ns_ref, o_ref):
    h = (a0_ref[...] + a1_ref[...]) * nd_ref[...] + b_ref[...]
    h = jnp.maximum(h, 0.0)
    o_ref[...] = jnp.dot(h, w_ref[...],
                         preferred_element_type=jnp.float32) * ns_ref[...]


_combine_mm = pl.pallas_call(
    _combine_mm_body,
    grid=(GR,),
    in_specs=[pl.BlockSpec((RB, FD), lambda i: (i, 0)),
              pl.BlockSpec((RB, FD), lambda i: (i, 0)),
              pl.BlockSpec((RB, 1), lambda i: (i, 0)),
              pl.BlockSpec((1, FD), lambda i: (0, 0)),
              pl.BlockSpec((FD, FD), lambda i: (0, 0)),
              pl.BlockSpec((RB, 1), lambda i: (i, 0))],
    out_specs=pl.BlockSpec((RB, FD), lambda i: (i, 0)),
    out_shape=jax.ShapeDtypeStruct((NPAD, FD), jnp.float32),
)


def _head_body(a0_ref, a1_ref, nd_ref, b_ref, pw_ref, f1w_ref, f1b_ref,
               f2w_ref, f2b_ref, o_ref, mx, sm):
    i = pl.program_id(0)
    h = (a0_ref[...] + a1_ref[...]) * nd_ref[...] + b_ref[...]
    h = jnp.maximum(h, 0.0)
    valid = lax.broadcasted_iota(jnp.int32, (RB, FD), 0) < (NN - i * RB)
    bm = jnp.max(jnp.where(valid, h, -1e30), axis=0, keepdims=True)
    bs = jnp.sum(jnp.where(valid, h, 0.0), axis=0, keepdims=True)

    @pl.when(i == 0)
    def _():
        mx[...] = bm
        sm[...] = bs

    @pl.when(i > 0)
    def _():
        mx[...] = jnp.maximum(mx[...], bm)
        sm[...] = sm[...] + bs

    @pl.when(i == GR - 1)
    def _():
        pwv = pw_ref[...]
        e = jnp.exp(pwv - jnp.max(pwv))
        wgt = e / jnp.sum(e)
        stacked = jnp.concatenate([mx[...], sm[...] * (1.0 / NN)], axis=0)
        rep = jnp.sum(stacked * wgt, axis=0, keepdims=True)
        x1 = jnp.dot(rep, f1w_ref[...],
                     preferred_element_type=jnp.float32) + f1b_ref[...]
        x1 = jnp.maximum(x1, 0.0)
        o_ref[...] = jnp.dot(x1, f2w_ref[...],
                             preferred_element_type=jnp.float32) + f2b_ref[...]


_head = pl.pallas_call(
    _head_body,
    grid=(GR,),
    in_specs=[pl.BlockSpec((RB, FD), lambda i: (i, 0)),
              pl.BlockSpec((RB, FD), lambda i: (i, 0)),
              pl.BlockSpec((RB, 1), lambda i: (i, 0)),
              pl.BlockSpec((1, FD), lambda i: (0, 0)),
              pl.BlockSpec((2, 1), lambda i: (0, 0)),
              pl.BlockSpec((FD, FD), lambda i: (0, 0)),
              pl.BlockSpec((1, FD), lambda i: (0, 0)),
              pl.BlockSpec((FD, 2), lambda i: (0, 0)),
              pl.BlockSpec((1, 2), lambda i: (0, 0))],
    out_specs=pl.BlockSpec((1, 2), lambda i: (0, 0)),
    out_shape=jax.ShapeDtypeStruct((1, 2), jnp.float32),
    scratch_shapes=[pltpu.VMEM((1, FD), jnp.float32),
                    pltpu.VMEM((1, FD), jnp.float32)],
)


_mesh = plsc.VectorSubcoreMesh(core_axis_name="c", subcore_axis_name="s")


@functools.partial(
    pl.kernel,
    out_type=jax.ShapeDtypeStruct((2, NPAD, FD), jnp.float32),
    mesh=_mesh,
    scratch_types=[
        pltpu.VMEM((SEG, CH), jnp.int32),
        pltpu.VMEM((SEG, CH), jnp.int32),
        pltpu.VMEM((CH, FD), jnp.float32),
        pltpu.VMEM((CH, FD), jnp.float32),
        pltpu.VMEM((16, FD), jnp.float32),
        pltpu.VMEM_SHARED((NPAD, FD), jnp.float32),
        pltpu.SemaphoreType.DMA,
        pltpu.SemaphoreType.DMA,
    ],
)
def _spmm(y_hbm, src_hbm, dst_hbm, z_hbm, out_hbm,
          sidx, didx, rows_a, rows_b, zbuf, agg_sh, sem_a, sem_b):
    cid = lax.axis_index("c")
    sid = lax.axis_index("s")
    wid = sid * 2 + cid

    pltpu.sync_copy(z_hbm, zbuf)
    r0 = sid * RPT
    for k in range(RPT // 16):
        pltpu.sync_copy(zbuf, agg_sh.at[pl.ds(r0 + k * 16, 16)])
    plsc.subcore_barrier()

    for seg in range(NCHUNK // SEG):
        pltpu.sync_copy(src_hbm.at[wid, pl.ds(seg * SEG, SEG)], sidx)
        pltpu.sync_copy(dst_hbm.at[wid, pl.ds(seg * SEG, SEG)], didx)
        pltpu.async_copy(y_hbm.at[sidx.at[0]], rows_a, sem_a)

        def step(u, carry):
            a = 2 * u
            pltpu.async_copy(y_hbm.at[sidx.at[a + 1]], rows_b, sem_b)
            pltpu.make_async_copy(y_hbm.at[sidx.at[0]], rows_a, sem_a).wait()
            pltpu.sync_copy(rows_a, agg_sh.at[didx.at[a]], add=True)

            @pl.when(u < SEG // 2 - 1)
            def _():
                pltpu.async_copy(y_hbm.at[sidx.at[a + 2]], rows_a, sem_a)

            pltpu.make_async_copy(y_hbm.at[sidx.at[0]], rows_b, sem_b).wait()
            pltpu.sync_copy(rows_b, agg_sh.at[didx.at[a + 1]], add=True)
            return carry

        lax.fori_loop(0, SEG // 2, step, 0)
    plsc.subcore_barrier()

    pltpu.sync_copy(agg_sh.at[pl.ds(r0, RPT)],
                    out_hbm.at[cid, pl.ds(r0, RPT)])


def kernel(features, edge_index, W1, b1, W2, b2, W3, b3, pool_weight,
           fc1_W, fc1_b, fc2_W, fc2_b):
    src = edge_index[0]
    dst = edge_index[1]
    padv = NN + (jnp.arange(EPAD - EE, dtype=jnp.int32) % (NPAD - NN))
    srcp = jnp.concatenate([src, padv])
    dstp = jnp.concatenate([dst, padv])
    src3 = srcp.reshape(NW, NCHUNK, CH)
    dst3 = dstp.reshape(NW, NCHUNK, CH)

    fpad = jnp.pad(features, ((0, NPAD - NN), (0, 0)))
    zeros_tile = jnp.zeros((16, FD), jnp.float32)

    ns2d, nd2d = _hist(srcp.reshape(EPAD, 1), dstp.reshape(EPAD, 1))
    ns = ns2d.reshape(NPAD, 1)
    nd = nd2d.reshape(NPAD, 1)

    y = _mm_scale(fpad, W1, ns)
    agg = _spmm(y, src3, dst3, zeros_tile)
    y = _combine_mm(agg[0], agg[1], nd, b1.reshape(1, FD), W2, ns)
    agg = _spmm(y, src3, dst3, zeros_tile)
    y = _combine_mm(agg[0], agg[1], nd, b2.reshape(1, FD), W3, ns)
    agg = _spmm(y, src3, dst3, zeros_tile)
    out = _head(agg[0], agg[1], nd, b3.reshape(1, FD),
                pool_weight.reshape(2, 1), fc1_W, fc1_b.reshape(1, FD),
                fc2_W, fc2_b.reshape(1, 2))
    return out


# speedup vs baseline: 2.8005x; 1.0363x over previous
import functools

import jax
import jax.numpy as jnp
from jax import lax
from jax.experimental import pallas as pl
from jax.experimental.pallas import tpu as pltpu
from jax.experimental.pallas import tpu_sc as plsc

NN = 10000
FD = 128
EE = 320000

NROW = 80
NPAD = NROW * 128
NW = 32
CH = 128
NCHUNK = 80
EPW = CH * NCHUNK
EPAD = NW * EPW
SEG = 16
EB = 8192
HG = EPAD // EB
RB = 2560
GR = NPAD // RB
RPT = NPAD // 16


def _hist_body(s_ref, d_ref, ns_ref, nd_ref, accs, accd):
    i = pl.program_id(0)
    iota_hi = lax.broadcasted_iota(jnp.int32, (1, NROW), 1)
    iota_lo = lax.broadcasted_iota(jnp.int32, (1, 128), 1)

    def onehot_dot(idx):
        hi = lax.shift_right_logical(idx, 7)
        lo = lax.bitwise_and(idx, 127)
        oh = jnp.where(hi == iota_hi, 1.0, 0.0)
        ol = jnp.where(lo == iota_lo, 1.0, 0.0)
        return lax.dot_general(oh, ol, (((0,), (0,)), ((), ())),
                               preferred_element_type=jnp.float32)

    cs = onehot_dot(s_ref[...])
    cd = onehot_dot(d_ref[...])

    @pl.when(i == 0)
    def _():
        accs[...] = cs
        accd[...] = cd

    @pl.when(i > 0)
    def _():
        accs[...] = accs[...] + cs
        accd[...] = accd[...] + cd

    @pl.when(i == HG - 1)
    def _():
        ns_ref[...] = lax.rsqrt(jnp.maximum(accs[...], 1.0))
        nd_ref[...] = lax.rsqrt(jnp.maximum(accd[...], 1.0))


_hist = pl.pallas_call(
    _hist_body,
    grid=(HG,),
    in_specs=[pl.BlockSpec((EB, 1), lambda i: (i, 0)),
              pl.BlockSpec((EB, 1), lambda i: (i, 0))],
    out_specs=[pl.BlockSpec((NROW, 128), lambda i: (0, 0)),
               pl.BlockSpec((NROW, 128), lambda i: (0, 0))],
    out_shape=[jax.ShapeDtypeStruct((NROW, 128), jnp.float32),
               jax.ShapeDtypeStruct((NROW, 128), jnp.float32)],
    scratch_shapes=[pltpu.VMEM((NROW, 128), jnp.float32),
                    pltpu.VMEM((NROW, 128), jnp.float32)],
)


def _mm_scale_body(x_ref, w_ref, ns_ref, o_ref):
    o_ref[...] = jnp.dot(x_ref[...], w_ref[...],
                         preferred_element_type=jnp.float32) * ns_ref[...]


_mm_scale = pl.pallas_call(
    _mm_scale_body,
    grid=(GR,),
    in_specs=[pl.BlockSpec((RB, FD), lambda i: (i, 0)),
              pl.BlockSpec((FD, FD), lambda i: (0, 0)),
              pl.BlockSpec((RB, 1), lambda i: (i, 0))],
    out_specs=pl.BlockSpec((RB, FD), lambda i: (i, 0)),
    out_shape=jax.ShapeDtypeStruct((NPAD, FD), jnp.float32),
)


def _combine_mm_body(a0_ref, a1_ref, nd_ref, b_ref, w_ref, ns_ref, o_ref):
    h = (a0_ref[...] + a1_ref[...]) * nd_ref[...] + b_ref[...]
    h = jnp.maximum(h, 0.0)
    o_ref[...] = jnp.dot(h, w_ref[...],
                         preferred_element_type=jnp.float32) * ns_ref[...]


_combine_mm = pl.pallas_call(
    _combine_mm_body,
    grid=(GR,),
    in_specs=[pl.BlockSpec((RB, FD), lambda i: (i, 0)),
              pl.BlockSpec((RB, FD), lambda i: (i, 0)),
              pl.BlockSpec((RB, 1), lambda i: (i, 0)),
              pl.BlockSpec((1, FD), lambda i: (0, 0)),
              pl.BlockSpec((FD, FD), lambda i: (0, 0)),
              pl.BlockSpec((RB, 1), lambda i: (i, 0))],
    out_specs=pl.BlockSpec((RB, FD), lambda i: (i, 0)),
    out_shape=jax.ShapeDtypeStruct((NPAD, FD), jnp.float32),
)


def _head_body(a0_ref, a1_ref, nd_ref, b_ref, pw_ref, f1w_ref, f1b_ref,
               f2w_ref, f2b_ref, o_ref, mx, sm):
    i = pl.program_id(0)
    h = (a0_ref[...] + a1_ref[...]) * nd_ref[...] + b_ref[...]
    h = jnp.maximum(h, 0.0)
    valid = lax.broadcasted_iota(jnp.int32, (RB, FD), 0) < (NN - i * RB)
    bm = jnp.max(jnp.where(valid, h, -1e30), axis=0, keepdims=True)
    bs = jnp.sum(jnp.where(valid, h, 0.0), axis=0, keepdims=True)

    @pl.when(i == 0)
    def _():
        mx[...] = bm
        sm[...] = bs

    @pl.when(i > 0)
    def _():
        mx[...] = jnp.maximum(mx[...], bm)
        sm[...] = sm[...] + bs

    @pl.when(i == GR - 1)
    def _():
        pwv = pw_ref[...]
        e = jnp.exp(pwv - jnp.max(pwv))
        wgt = e / jnp.sum(e)
        stacked = jnp.concatenate([mx[...], sm[...] * (1.0 / NN)], axis=0)
        rep = jnp.sum(stacked * wgt, axis=0, keepdims=True)
        x1 = jnp.dot(rep, f1w_ref[...],
                     preferred_element_type=jnp.float32) + f1b_ref[...]
        x1 = jnp.maximum(x1, 0.0)
        o_ref[...] = jnp.dot(x1, f2w_ref[...],
                             preferred_element_type=jnp.float32) + f2b_ref[...]


_head = pl.pallas_call(
    _head_body,
    grid=(GR,),
    in_specs=[pl.BlockSpec((RB, FD), lambda i: (i, 0)),
              pl.BlockSpec((RB, FD), lambda i: (i, 0)),
              pl.BlockSpec((RB, 1), lambda i: (i, 0)),
              pl.BlockSpec((1, FD), lambda i: (0, 0)),
              pl.BlockSpec((2, 1), lambda i: (0, 0)),
              pl.BlockSpec((FD, FD), lambda i: (0, 0)),
              pl.BlockSpec((1, FD), lambda i: (0, 0)),
              pl.BlockSpec((FD, 2), lambda i: (0, 0)),
              pl.BlockSpec((1, 2), lambda i: (0, 0))],
    out_specs=pl.BlockSpec((1, 2), lambda i: (0, 0)),
    out_shape=jax.ShapeDtypeStruct((1, 2), jnp.float32),
    scratch_shapes=[pltpu.VMEM((1, FD), jnp.float32),
                    pltpu.VMEM((1, FD), jnp.float32)],
)


_mesh = plsc.VectorSubcoreMesh(core_axis_name="c", subcore_axis_name="s")


@functools.partial(
    pl.kernel,
    out_type=jax.ShapeDtypeStruct((2, NPAD, FD), jnp.float32),
    mesh=_mesh,
    scratch_types=[
        pltpu.VMEM((SEG, CH), jnp.int32),
        pltpu.VMEM((SEG, CH), jnp.int32),
        pltpu.VMEM((CH, FD), jnp.float32),
        pltpu.VMEM((CH, FD), jnp.float32),
        pltpu.VMEM((16, FD), jnp.float32),
        pltpu.VMEM_SHARED((NPAD, FD), jnp.float32),
        pltpu.SemaphoreType.DMA,
        pltpu.SemaphoreType.DMA,
    ],
)
def _spmm(y_hbm, src_hbm, dst_hbm, z_hbm, out_hbm,
          sidx, didx, rows_a, rows_b, zbuf, agg_sh, sem_a, sem_b):
    cid = lax.axis_index("c")
    sid = lax.axis_index("s")
    wid = sid * 2 + cid

    pltpu.sync_copy(z_hbm, zbuf)
    r0 = sid * RPT
    for k in range(RPT // 16):
        pltpu.sync_copy(zbuf, agg_sh.at[pl.ds(r0 + k * 16, 16)])
    plsc.subcore_barrier()

    for seg in range(NCHUNK // SEG):
        pltpu.sync_copy(src_hbm.at[wid, pl.ds(seg * SEG, SEG)], sidx)
        pltpu.sync_copy(dst_hbm.at[wid, pl.ds(seg * SEG, SEG)], didx)
        pltpu.async_copy(y_hbm.at[sidx.at[0]], rows_a, sem_a)

        def step(u, carry):
            a = 2 * u
            pltpu.async_copy(y_hbm.at[sidx.at[a + 1]], rows_b, sem_b)
            pltpu.make_async_copy(y_hbm.at[sidx.at[0]], rows_a, sem_a).wait()
            pltpu.sync_copy(rows_a, agg_sh.at[didx.at[a]], add=True)

            @pl.when(u < SEG // 2 - 1)
            def _():
                pltpu.async_copy(y_hbm.at[sidx.at[a + 2]], rows_a, sem_a)

            pltpu.make_async_copy(y_hbm.at[sidx.at[0]], rows_b, sem_b).wait()
            pltpu.sync_copy(rows_b, agg_sh.at[didx.at[a + 1]], add=True)
            return carry

        lax.fori_loop(0, SEG // 2, step, 0)
    plsc.subcore_barrier()

    pltpu.sync_copy(agg_sh.at[pl.ds(r0, RPT)],
                    out_hbm.at[cid, pl.ds(r0, RPT)])


def kernel(features, edge_index, W1, b1, W2, b2, W3, b3, pool_weight,
           fc1_W, fc1_b, fc2_W, fc2_b):
    src = edge_index[0]
    dst = edge_index[1]
    padv = NN + (jnp.arange(EPAD - EE, dtype=jnp.int32) % (NPAD - NN))
    srcp = jnp.concatenate([src, padv])
    dstp = jnp.concatenate([dst, padv])
    src3 = srcp.reshape(NW, NCHUNK, CH)
    dst3 = dstp.reshape(NW, NCHUNK, CH)

    fpad = jnp.pad(features, ((0, NPAD - NN), (0, 0)))
    zeros_tile = jnp.zeros((16, FD), jnp.float32)

    ns2d, nd2d = _hist(srcp.reshape(EPAD, 1), dstp.reshape(EPAD, 1))
    ns = ns2d.reshape(NPAD, 1)
    nd = nd2d.reshape(NPAD, 1)

    y = _mm_scale(fpad, W1, ns)
    agg = _spmm(y, src3, dst3, zeros_tile)
    y = _combine_mm(agg[0], agg[1], nd, b1.reshape(1, FD), W2, ns)
    agg = _spmm(y, src3, dst3, zeros_tile)
    y = _combine_mm(agg[0], agg[1], nd, b2.reshape(1, FD), W3, ns)
    agg = _spmm(y, src3, dst3, zeros_tile)
    out = _head(agg[0], agg[1], nd, b3.reshape(1, FD),
                pool_weight.reshape(2, 1), fc1_W, fc1_b.reshape(1, FD),
                fc2_W, fc2_b.reshape(1, 2))
    return out
